# Initial kernel scaffold; baseline (speedup 1.0000x reference)
import jax
import jax.numpy as jnp
from jax.experimental import pallas as pl


def kernel(x, h, c, edge_index, Wfu, bfu, Wfw, bfw, Wiu, biu, Wiw, biw, Wou, bou, Wow, bow, U):
    raise NotImplementedError("write your pallas kernel here")


# speedup vs baseline: 13.0857x; 13.0857x over previous
import functools

import jax
import jax.numpy as jnp
from jax import lax
from jax.experimental import pallas as pl
from jax.experimental.pallas import tpu as pltpu
from jax.experimental.pallas import tpu_sc as plsc

_LANES = 128
_NS = 16
_NC = 2
_JM = 2
_JD = 16


def _round_up(a, b):
    return (a + b - 1) // b * b


def _sc_mesh():
    return plsc.VectorSubcoreMesh(core_axis_name="c", subcore_axis_name="s")


def _deg_body(rt, gd, kd, dst2d, init2, ones_h, out, didx, ones_v, acc):
    c = lax.axis_index("c")
    s = lax.axis_index("s")
    wid = c * _NS + s
    rb = s * rt
    pltpu.sync_copy(init2.at[c, pl.ds(rb, rt)], acc.at[pl.ds(rb, rt)])
    pltpu.sync_copy(ones_h, ones_v)
    plsc.subcore_barrier()

    def outer(k, carry):
        g0 = wid * gd + k * _JD
        pltpu.sync_copy(dst2d.at[pl.ds(g0, _JD)], didx)
        for j in range(_JD):
            pltpu.sync_copy(ones_v, acc.at[didx.at[j]], add=True)
        return carry

    lax.fori_loop(0, kd, outer, 0)
    plsc.subcore_barrier()
    pltpu.sync_copy(acc.at[pl.ds(rb, rt)], out.at[c, pl.ds(rb, rt)])


def _msg_body(rt, km_full, km_half,
              src2d, dst2d, zf, zi, zo, zzero, sf, si, so0, so1,
              sidx, didx, buf, acc):
    c = lax.axis_index("c")
    s = lax.axis_index("s")
    rb = s * rt

    def run(z_h, init_h, out_h, gbase, km):
        pltpu.sync_copy(init_h.at[pl.ds(rb, rt)], acc.at[pl.ds(rb, rt)])
        plsc.subcore_barrier()

        def outer(k, carry):
            g0 = gbase + s * (km * _JM) + k * _JM
            pltpu.sync_copy(src2d.at[pl.ds(g0, _JM)], sidx)
            for j in range(_JM):
                pltpu.sync_copy(z_h.at[sidx.at[j]],
                                buf.at[pl.ds(j * _LANES, _LANES)])
            pltpu.sync_copy(dst2d.at[pl.ds(g0, _JM)], didx)
            for j in range(_JM):
                pltpu.sync_copy(buf.at[pl.ds(j * _LANES, _LANES)],
                                acc.at[didx.at[j]], add=True)
            return carry

        lax.fori_loop(0, km, outer, 0)
        plsc.subcore_barrier()
        pltpu.sync_copy(acc.at[pl.ds(rb, rt)], out_h.at[pl.ds(rb, rt)])

    half_g = _NS * km_half * _JM

    @pl.when(c == 0)
    def _():
        run(zf, zf, sf, 0, km_full)
        run(zo, zo, so0, 0, km_half)

    @pl.when(c == 1)
    def _():
        run(zi, zi, si, 0, km_full)
        run(zo, zzero, so1, half_g, km_half)


def _tc12_body(xref, href, dref, wuref, wwref, uref, zfref, ziref, zoref, tref):
    d = dref[0, :, 0:1] + dref[1, :, 0:1]
    dinv = lax.rsqrt(d)
    z = (jnp.dot(xref[...], wuref[...], preferred_element_type=jnp.float32)
         + jnp.dot(href[...], wwref[...], preferred_element_type=jnp.float32))
    zs = z * dinv
    dd = xref.shape[1]
    zfref[...] = zs[:, :dd]
    ziref[...] = zs[:, dd:2 * dd]
    zoref[...] = zs[:, 2 * dd:3 * dd]
    tref[...] = jnp.tanh(jnp.dot(xref[...], uref[...],
                                 preferred_element_type=jnp.float32))


def _tc3_body(sfref, siref, so0ref, so1ref, dref, tref, cref, bref,
              hout, cout):
    d = dref[0, :, 0:1] + dref[1, :, 0:1]
    dinv = lax.rsqrt(d)
    dd = tref.shape[1]
    bf = bref[0:1, :dd]
    bi = bref[0:1, dd:2 * dd]
    bo = bref[0:1, 2 * dd:3 * dd]
    f = jax.nn.sigmoid(sfref[...] * dinv + bf)
    i = jax.nn.sigmoid((siref[...] * dinv + bi) * tref[...])
    o = jax.nn.sigmoid((so0ref[...] + so1ref[...]) * dinv + bo)
    cn = cref[...] * f + i
    cout[...] = cn
    hout[...] = o * jnp.tanh(cn)


def kernel(x, h, c, edge_index, Wfu, bfu, Wfw, bfw, Wiu, biu, Wiw, biw,
           Wou, bou, Wow, bow, U):
    f32 = jnp.float32
    n, d = x.shape
    e = edge_index.shape[1]
    npad = _round_up(n + 1, _NS * 80)
    rt = npad // _NS
    epad = _round_up(e, _LANES * _NS * _JM * _NC)
    g = epad // _LANES
    km_full = g // (_NS * _JM)
    km_half = km_full // 2
    gd = g // (_NS * _NC)
    kd = gd // _JD

    wu = jnp.concatenate([Wfu, Wiu, Wou], axis=1)
    ww = jnp.concatenate([Wfw, Wiw, Wow], axis=1)
    bcat = jnp.concatenate([bfu + bfw, biu + biw, bou + bow])[None, :]
    xp = jnp.zeros((npad, d), f32).at[:n, :].set(x)
    hp = jnp.zeros((npad, d), f32).at[:n, :].set(h)
    pad = jnp.full((epad - e,), n, jnp.int32)
    src2d = jnp.concatenate([edge_index[0], pad]).reshape(g, _LANES)
    dst2d = jnp.concatenate([edge_index[1], pad]).reshape(g, _LANES)
    init2 = jnp.zeros((_NC, npad, _LANES), f32).at[0, :, 0].set(1.0)
    ones_h = jnp.zeros((_LANES, _LANES), f32).at[:, 0].set(1.0)
    zzero = jnp.zeros((npad, d), f32)

    dcnt = pl.kernel(
        functools.partial(_deg_body, rt, gd, kd),
        out_type=jax.ShapeDtypeStruct((_NC, npad, _LANES), f32),
        mesh=_sc_mesh(),
        scratch_types=[
            pltpu.VMEM((_JD, _LANES), jnp.int32),
            pltpu.VMEM((_LANES, _LANES), f32),
            pltpu.VMEM_SHARED((npad, _LANES), f32),
        ],
    )(dst2d, init2, ones_h)

    blk = npad // 10
    zf, zi, zo, t = pl.pallas_call(
        _tc12_body,
        grid=(npad // blk,),
        in_specs=[
            pl.BlockSpec((blk, d), lambda i: (i, 0)),
            pl.BlockSpec((blk, d), lambda i: (i, 0)),
            pl.BlockSpec((_NC, blk, _LANES), lambda i: (0, i, 0)),
            pl.BlockSpec((d, 3 * d), lambda i: (0, 0)),
            pl.BlockSpec((d, 3 * d), lambda i: (0, 0)),
            pl.BlockSpec((d, d), lambda i: (0, 0)),
        ],
        out_specs=[
            pl.BlockSpec((blk, d), lambda i: (i, 0)),
            pl.BlockSpec((blk, d), lambda i: (i, 0)),
            pl.BlockSpec((blk, d), lambda i: (i, 0)),
            pl.BlockSpec((blk, d), lambda i: (i, 0)),
        ],
        out_shape=[
            jax.ShapeDtypeStruct((npad, d), f32),
            jax.ShapeDtypeStruct((npad, d), f32),
            jax.ShapeDtypeStruct((npad, d), f32),
            jax.ShapeDtypeStruct((npad, d), f32),
        ],
    )(xp, hp, dcnt, wu, ww, U)

    sf, si, so0, so1 = pl.kernel(
        functools.partial(_msg_body, rt, km_full, km_half),
        out_type=(jax.ShapeDtypeStruct((npad, d), f32),
                  jax.ShapeDtypeStruct((npad, d), f32),
                  jax.ShapeDtypeStruct((npad, d), f32),
                  jax.ShapeDtypeStruct((npad, d), f32)),
        mesh=_sc_mesh(),
        scratch_types=[
            pltpu.VMEM((_JM, _LANES), jnp.int32),
            pltpu.VMEM((_JM, _LANES), jnp.int32),
            pltpu.VMEM((_JM * _LANES, d), f32),
            pltpu.VMEM_SHARED((npad, d), f32),
        ],
    )(src2d, dst2d, zf, zi, zo, zzero)

    blk2 = n // 10
    hn, cn = pl.pallas_call(
        _tc3_body,
        grid=(n // blk2,),
        in_specs=[
            pl.BlockSpec((blk2, d), lambda i: (i, 0)),
            pl.BlockSpec((blk2, d), lambda i: (i, 0)),
            pl.BlockSpec((blk2, d), lambda i: (i, 0)),
            pl.BlockSpec((blk2, d), lambda i: (i, 0)),
            pl.BlockSpec((_NC, blk2, _LANES), lambda i: (0, i, 0)),
            pl.BlockSpec((blk2, d), lambda i: (i, 0)),
            pl.BlockSpec((blk2, d), lambda i: (i, 0)),
            pl.BlockSpec((1, 3 * d), lambda i: (0, 0)),
        ],
        out_specs=[
            pl.BlockSpec((blk2, d), lambda i: (i, 0)),
            pl.BlockSpec((blk2, d), lambda i: (i, 0)),
        ],
        out_shape=[
            jax.ShapeDtypeStruct((n, d), f32),
            jax.ShapeDtypeStruct((n, d), f32),
        ],
    )(sf, si, so0, so1, dcnt, t, c, bcat)

    return (hn, cn)


# speedup vs baseline: 15.3948x; 1.1765x over previous
import functools

import jax
import jax.numpy as jnp
from jax import lax
from jax.experimental import pallas as pl
from jax.experimental.pallas import tpu as pltpu
from jax.experimental.pallas import tpu_sc as plsc

_LANES = 128
_NS = 16
_NC = 2
_GB = 16


def _round_up(a, b):
    return (a + b - 1) // b * b


def _sc_mesh():
    return plsc.VectorSubcoreMesh(core_axis_name="c", subcore_axis_name="s")


def _deg_body(rt, gd, dst2d, init2, ones_h, out, didx, ones_v, acc, sem):
    c = lax.axis_index("c")
    s = lax.axis_index("s")
    wid = c * _NS + s
    rb = s * rt
    pltpu.sync_copy(init2.at[c, pl.ds(rb, rt)], acc.at[pl.ds(rb, rt)])
    pltpu.sync_copy(ones_h, ones_v)
    plsc.subcore_barrier()

    def batch(bi, carry):
        g0 = wid * gd + bi * _GB
        pltpu.sync_copy(dst2d.at[pl.ds(g0, _GB)], didx)
        for j in range(_GB):
            pltpu.async_copy(ones_v, acc.at[didx.at[j]], sem, add=True)
        for j in range(_GB):
            pltpu.make_async_copy(ones_v, acc.at[didx.at[j]], sem).wait()
        return carry

    lax.fori_loop(0, gd // _GB, batch, 0)
    plsc.subcore_barrier()
    pltpu.sync_copy(acc.at[pl.ds(rb, rt)], out.at[c, pl.ds(rb, rt)])


def _msg_body(rt, km_full, km_half,
              src2d, dst2d, zf, zi, zo, zzero, sf, si, so0, so1,
              sidx, didx, buf0, buf1, acc, gsem0, gsem1, ssem0, ssem1):
    c = lax.axis_index("c")
    s = lax.axis_index("s")
    rb = s * rt
    bufs = (buf0, buf1)
    gsems = (gsem0, gsem1)
    ssems = (ssem0, ssem1)

    def run(z_h, init_h, out_h, gbase, km):
        pltpu.sync_copy(init_h.at[pl.ds(rb, rt)], acc.at[pl.ds(rb, rt)])
        plsc.subcore_barrier()
        tg0 = gbase + s * km

        def batch(bi, carry):
            g0 = tg0 + bi * _GB
            pltpu.sync_copy(src2d.at[pl.ds(g0, _GB)], sidx)
            pltpu.sync_copy(dst2d.at[pl.ds(g0, _GB)], didx)
            pltpu.async_copy(z_h.at[sidx.at[0]], bufs[0], gsems[0])
            for j in range(_GB):
                b = j % 2
                pltpu.make_async_copy(z_h.at[sidx.at[j]], bufs[b],
                                      gsems[b]).wait()
                pltpu.async_copy(bufs[b], acc.at[didx.at[j]], ssems[b],
                                 add=True)
                if j + 1 < _GB:
                    nb = (j + 1) % 2
                    if j >= 1:
                        pltpu.make_async_copy(bufs[nb],
                                              acc.at[didx.at[j - 1]],
                                              ssems[nb]).wait()
                    pltpu.async_copy(z_h.at[sidx.at[j + 1]], bufs[nb],
                                     gsems[nb])
            pltpu.make_async_copy(bufs[(_GB - 2) % 2],
                                  acc.at[didx.at[_GB - 2]],
                                  ssems[(_GB - 2) % 2]).wait()
            pltpu.make_async_copy(bufs[(_GB - 1) % 2],
                                  acc.at[didx.at[_GB - 1]],
                                  ssems[(_GB - 1) % 2]).wait()
            return carry

        lax.fori_loop(0, km // _GB, batch, 0)
        plsc.subcore_barrier()
        pltpu.sync_copy(acc.at[pl.ds(rb, rt)], out_h.at[pl.ds(rb, rt)])

    half_g = _NS * km_half

    @pl.when(c == 0)
    def _():
        run(zf, zf, sf, 0, km_full)
        run(zo, zo, so0, 0, km_half)

    @pl.when(c == 1)
    def _():
        run(zi, zi, si, 0, km_full)
        run(zo, zzero, so1, half_g, km_half)


def _tc12_body(xref, href, dref, wuref, wwref, uref, zfref, ziref, zoref, tref):
    d = dref[0, :, 0:1] + dref[1, :, 0:1]
    dinv = lax.rsqrt(d)
    z = (jnp.dot(xref[...], wuref[...], preferred_element_type=jnp.float32)
         + jnp.dot(href[...], wwref[...], preferred_element_type=jnp.float32))
    zs = z * dinv
    dd = xref.shape[1]
    zfref[...] = zs[:, :dd]
    ziref[...] = zs[:, dd:2 * dd]
    zoref[...] = zs[:, 2 * dd:3 * dd]
    tref[...] = jnp.tanh(jnp.dot(xref[...], uref[...],
                                 preferred_element_type=jnp.float32))


def _tc3_body(sfref, siref, so0ref, so1ref, dref, tref, cref, bref,
              hout, cout):
    d = dref[0, :, 0:1] + dref[1, :, 0:1]
    dinv = lax.rsqrt(d)
    dd = tref.shape[1]
    bf = bref[0:1, :dd]
    bi = bref[0:1, dd:2 * dd]
    bo = bref[0:1, 2 * dd:3 * dd]
    f = jax.nn.sigmoid(sfref[...] * dinv + bf)
    i = jax.nn.sigmoid((siref[...] * dinv + bi) * tref[...])
    o = jax.nn.sigmoid((so0ref[...] + so1ref[...]) * dinv + bo)
    cn = cref[...] * f + i
    cout[...] = cn
    hout[...] = o * jnp.tanh(cn)


def kernel(x, h, c, edge_index, Wfu, bfu, Wfw, bfw, Wiu, biu, Wiw, biw,
           Wou, bou, Wow, bow, U):
    f32 = jnp.float32
    n, d = x.shape
    e = edge_index.shape[1]
    npad = _round_up(n + 1, _NS * 80)
    rt = npad // _NS
    epad = _round_up(e, _LANES * _NS * 2 * _GB)
    g = epad // _LANES
    km_full = g // _NS
    km_half = km_full // 2
    gd = g // (_NS * _NC)

    wu = jnp.concatenate([Wfu, Wiu, Wou], axis=1)
    ww = jnp.concatenate([Wfw, Wiw, Wow], axis=1)
    bcat = jnp.concatenate([bfu + bfw, biu + biw, bou + bow])[None, :]
    xp = jnp.zeros((npad, d), f32).at[:n, :].set(x)
    hp = jnp.zeros((npad, d), f32).at[:n, :].set(h)
    pad = jnp.full((epad - e,), n, jnp.int32)
    src2d = jnp.concatenate([edge_index[0], pad]).reshape(g, _LANES)
    dst2d = jnp.concatenate([edge_index[1], pad]).reshape(g, _LANES)
    init2 = jnp.zeros((_NC, npad, _LANES), f32).at[0, :, 0].set(1.0)
    ones_h = jnp.zeros((_LANES, _LANES), f32).at[:, 0].set(1.0)
    zzero = jnp.zeros((npad, d), f32)

    dcnt = pl.kernel(
        functools.partial(_deg_body, rt, gd),
        out_type=jax.ShapeDtypeStruct((_NC, npad, _LANES), f32),
        mesh=_sc_mesh(),
        scratch_types=[
            pltpu.VMEM((_GB, _LANES), jnp.int32),
            pltpu.VMEM((_LANES, _LANES), f32),
            pltpu.VMEM_SHARED((npad, _LANES), f32),
            pltpu.SemaphoreType.DMA,
        ],
    )(dst2d, init2, ones_h)

    blk = npad // 10
    zf, zi, zo, t = pl.pallas_call(
        _tc12_body,
        grid=(npad // blk,),
        in_specs=[
            pl.BlockSpec((blk, d), lambda i: (i, 0)),
            pl.BlockSpec((blk, d), lambda i: (i, 0)),
            pl.BlockSpec((_NC, blk, _LANES), lambda i: (0, i, 0)),
            pl.BlockSpec((d, 3 * d), lambda i: (0, 0)),
            pl.BlockSpec((d, 3 * d), lambda i: (0, 0)),
            pl.BlockSpec((d, d), lambda i: (0, 0)),
        ],
        out_specs=[
            pl.BlockSpec((blk, d), lambda i: (i, 0)),
            pl.BlockSpec((blk, d), lambda i: (i, 0)),
            pl.BlockSpec((blk, d), lambda i: (i, 0)),
            pl.BlockSpec((blk, d), lambda i: (i, 0)),
        ],
        out_shape=[
            jax.ShapeDtypeStruct((npad, d), f32),
            jax.ShapeDtypeStruct((npad, d), f32),
            jax.ShapeDtypeStruct((npad, d), f32),
            jax.ShapeDtypeStruct((npad, d), f32),
        ],
    )(xp, hp, dcnt, wu, ww, U)

    sf, si, so0, so1 = pl.kernel(
        functools.partial(_msg_body, rt, km_full, km_half),
        out_type=(jax.ShapeDtypeStruct((npad, d), f32),
                  jax.ShapeDtypeStruct((npad, d), f32),
                  jax.ShapeDtypeStruct((npad, d), f32),
                  jax.ShapeDtypeStruct((npad, d), f32)),
        mesh=_sc_mesh(),
        scratch_types=[
            pltpu.VMEM((_GB, _LANES), jnp.int32),
            pltpu.VMEM((_GB, _LANES), jnp.int32),
            pltpu.VMEM((_LANES, d), f32),
            pltpu.VMEM((_LANES, d), f32),
            pltpu.VMEM_SHARED((npad, d), f32),
            pltpu.SemaphoreType.DMA,
            pltpu.SemaphoreType.DMA,
            pltpu.SemaphoreType.DMA,
            pltpu.SemaphoreType.DMA,
        ],
    )(src2d, dst2d, zf, zi, zo, zzero)

    blk2 = n // 10
    hn, cn = pl.pallas_call(
        _tc3_body,
        grid=(n // blk2,),
        in_specs=[
            pl.BlockSpec((blk2, d), lambda i: (i, 0)),
            pl.BlockSpec((blk2, d), lambda i: (i, 0)),
            pl.BlockSpec((blk2, d), lambda i: (i, 0)),
            pl.BlockSpec((blk2, d), lambda i: (i, 0)),
            pl.BlockSpec((_NC, blk2, _LANES), lambda i: (0, i, 0)),
            pl.BlockSpec((blk2, d), lambda i: (i, 0)),
            pl.BlockSpec((blk2, d), lambda i: (i, 0)),
            pl.BlockSpec((1, 3 * d), lambda i: (0, 0)),
        ],
        out_specs=[
            pl.BlockSpec((blk2, d), lambda i: (i, 0)),
            pl.BlockSpec((blk2, d), lambda i: (i, 0)),
        ],
        out_shape=[
            jax.ShapeDtypeStruct((n, d), f32),
            jax.ShapeDtypeStruct((n, d), f32),
        ],
    )(sf, si, so0, so1, dcnt, t, c, bcat)

    return (hn, cn)


# speedup vs baseline: 37.9239x; 2.4634x over previous
import functools

import jax
import jax.numpy as jnp
from jax import lax
from jax.experimental import pallas as pl
from jax.experimental.pallas import tpu as pltpu
from jax.experimental.pallas import tpu_sc as plsc

_LANES = 128
_NS = 16
_NC = 2
_GB = 16


def _round_up(a, b):
    return (a + b - 1) // b * b


def _sc_mesh():
    return plsc.VectorSubcoreMesh(core_axis_name="c", subcore_axis_name="s")


def _deg_body(rt, gd, dst2d, init2, ones_h, out, didx, ones_v, acc, sem):
    c = lax.axis_index("c")
    s = lax.axis_index("s")
    wid = c * _NS + s
    rb = s * rt
    pltpu.sync_copy(init2.at[c, pl.ds(rb, rt)], acc.at[pl.ds(rb, rt)])
    pltpu.sync_copy(ones_h, ones_v)
    plsc.subcore_barrier()

    def batch(bi, carry):
        g0 = wid * gd + bi * _GB
        pltpu.sync_copy(dst2d.at[pl.ds(g0, _GB)], didx)
        for j in range(_GB):
            pltpu.async_copy(ones_v, acc.at[didx.at[j]], sem, add=True)
        for j in range(_GB):
            pltpu.make_async_copy(ones_v, acc.at[didx.at[j]], sem).wait()
        return carry

    lax.fori_loop(0, gd // _GB, batch, 0)
    plsc.subcore_barrier()
    pltpu.sync_copy(acc.at[pl.ds(rb, rt)], out.at[c, pl.ds(rb, rt)])


_DEPTH = 4
_SUB = 64


def _msg_body(rt, km_half,
              src2d, dst2d, zf, zi, zo, zzero,
              sf0, sf1, si0, si1, so0, so1,
              sidx, didx, buf0, buf1, buf2, buf3, acc,
              gsem0, gsem1, gsem2, gsem3, ssem0, ssem1, ssem2, ssem3):
    c = lax.axis_index("c")
    s = lax.axis_index("s")
    rb = s * rt
    bufs = (buf0, buf1, buf2, buf3)
    gsems = (gsem0, gsem1, gsem2, gsem3)
    ssems = (ssem0, ssem1, ssem2, ssem3)
    nq = 2 * _GB

    def run(z_h, init_h, out_h, gbase):
        pltpu.sync_copy(init_h.at[pl.ds(rb, rt)], acc.at[pl.ds(rb, rt)])
        plsc.subcore_barrier()
        tg0 = gbase + s * km_half

        def src_sl(q):
            return z_h.at[sidx.at[q // 2, pl.ds((q % 2) * _SUB, _SUB)]]

        def dst_sl(q):
            return acc.at[didx.at[q // 2, pl.ds((q % 2) * _SUB, _SUB)]]

        def batch(bi, carry):
            g0 = tg0 + bi * _GB
            pltpu.sync_copy(src2d.at[pl.ds(g0, _GB)], sidx)
            pltpu.sync_copy(dst2d.at[pl.ds(g0, _GB)], didx)
            for q in range(_DEPTH - 1):
                pltpu.async_copy(src_sl(q), bufs[q], gsems[q])
            for q in range(nq):
                b = q % _DEPTH
                pltpu.make_async_copy(src_sl(q), bufs[b], gsems[b]).wait()
                pltpu.async_copy(bufs[b], dst_sl(q), ssems[b], add=True)
                if q + _DEPTH - 1 < nq:
                    pb = (q + _DEPTH - 1) % _DEPTH
                    if q >= 1:
                        pltpu.make_async_copy(bufs[pb], dst_sl(q - 1),
                                              ssems[pb]).wait()
                    pltpu.async_copy(src_sl(q + _DEPTH - 1), bufs[pb],
                                     gsems[pb])
            for q in range(nq - _DEPTH, nq):
                b = q % _DEPTH
                pltpu.make_async_copy(bufs[b], dst_sl(q), ssems[b]).wait()
            return carry

        lax.fori_loop(0, km_half // _GB, batch, 0)
        plsc.subcore_barrier()
        pltpu.sync_copy(acc.at[pl.ds(rb, rt)], out_h.at[pl.ds(rb, rt)])

    half_g = _NS * km_half

    @pl.when(c == 0)
    def _():
        run(zf, zf, sf0, 0)
        run(zi, zzero, si1, half_g)
        run(zo, zo, so0, 0)

    @pl.when(c == 1)
    def _():
        run(zi, zi, si0, 0)
        run(zf, zzero, sf1, half_g)
        run(zo, zzero, so1, half_g)


def _tc12_body(xref, href, dref, wuref, wwref, uref, zfref, ziref, zoref, tref):
    d = dref[0, :, 0:1] + dref[1, :, 0:1]
    dinv = lax.rsqrt(d)
    z = (jnp.dot(xref[...], wuref[...], preferred_element_type=jnp.float32)
         + jnp.dot(href[...], wwref[...], preferred_element_type=jnp.float32))
    zs = z * dinv
    dd = xref.shape[1]
    zfref[...] = zs[:, :dd]
    ziref[...] = zs[:, dd:2 * dd]
    zoref[...] = zs[:, 2 * dd:3 * dd]
    tref[...] = jnp.tanh(jnp.dot(xref[...], uref[...],
                                 preferred_element_type=jnp.float32))


def _tc3_body(sf0ref, sf1ref, si0ref, si1ref, so0ref, so1ref,
              dref, tref, cref, bref, hout, cout):
    d = dref[0, :, 0:1] + dref[1, :, 0:1]
    dinv = lax.rsqrt(d)
    dd = tref.shape[1]
    bf = bref[0:1, :dd]
    bi = bref[0:1, dd:2 * dd]
    bo = bref[0:1, 2 * dd:3 * dd]
    f = jax.nn.sigmoid((sf0ref[...] + sf1ref[...]) * dinv + bf)
    i = jax.nn.sigmoid(((si0ref[...] + si1ref[...]) * dinv + bi) * tref[...])
    o = jax.nn.sigmoid((so0ref[...] + so1ref[...]) * dinv + bo)
    cn = cref[...] * f + i
    cout[...] = cn
    hout[...] = o * jnp.tanh(cn)


def kernel(x, h, c, edge_index, Wfu, bfu, Wfw, bfw, Wiu, biu, Wiw, biw,
           Wou, bou, Wow, bow, U):
    f32 = jnp.float32
    n, d = x.shape
    e = edge_index.shape[1]
    npad = _round_up(n + 1, _NS * 80)
    rt = npad // _NS
    epad = _round_up(e, _LANES * _NS * 2 * _GB)
    g = epad // _LANES
    km_half = g // (2 * _NS)
    gd = g // (_NS * _NC)

    wu = jnp.concatenate([Wfu, Wiu, Wou], axis=1)
    ww = jnp.concatenate([Wfw, Wiw, Wow], axis=1)
    bcat = jnp.concatenate([bfu + bfw, biu + biw, bou + bow])[None, :]
    xp = jnp.zeros((npad, d), f32).at[:n, :].set(x)
    hp = jnp.zeros((npad, d), f32).at[:n, :].set(h)
    pidx = jnp.arange(epad - e, dtype=jnp.int32)
    pad_src = (pidx * 131) % n
    pad_dst = n + (pidx % (npad - n))
    src2d = jnp.concatenate([edge_index[0], pad_src]).reshape(g, _LANES)
    dst2d = jnp.concatenate([edge_index[1], pad_dst]).reshape(g, _LANES)
    init2 = jnp.zeros((_NC, npad, _LANES), f32).at[0, :, 0].set(1.0)
    ones_h = jnp.zeros((_LANES, _LANES), f32).at[:, 0].set(1.0)
    zzero = jnp.zeros((npad, d), f32)

    dcnt = pl.kernel(
        functools.partial(_deg_body, rt, gd),
        out_type=jax.ShapeDtypeStruct((_NC, npad, _LANES), f32),
        mesh=_sc_mesh(),
        scratch_types=[
            pltpu.VMEM((_GB, _LANES), jnp.int32),
            pltpu.VMEM((_LANES, _LANES), f32),
            pltpu.VMEM_SHARED((npad, _LANES), f32),
            pltpu.SemaphoreType.DMA,
        ],
    )(dst2d, init2, ones_h)

    blk = npad // 10
    zf, zi, zo, t = pl.pallas_call(
        _tc12_body,
        grid=(npad // blk,),
        in_specs=[
            pl.BlockSpec((blk, d), lambda i: (i, 0)),
            pl.BlockSpec((blk, d), lambda i: (i, 0)),
            pl.BlockSpec((_NC, blk, _LANES), lambda i: (0, i, 0)),
            pl.BlockSpec((d, 3 * d), lambda i: (0, 0)),
            pl.BlockSpec((d, 3 * d), lambda i: (0, 0)),
            pl.BlockSpec((d, d), lambda i: (0, 0)),
        ],
        out_specs=[
            pl.BlockSpec((blk, d), lambda i: (i, 0)),
            pl.BlockSpec((blk, d), lambda i: (i, 0)),
            pl.BlockSpec((blk, d), lambda i: (i, 0)),
            pl.BlockSpec((blk, d), lambda i: (i, 0)),
        ],
        out_shape=[
            jax.ShapeDtypeStruct((npad, d), f32),
            jax.ShapeDtypeStruct((npad, d), f32),
            jax.ShapeDtypeStruct((npad, d), f32),
            jax.ShapeDtypeStruct((npad, d), f32),
        ],
    )(xp, hp, dcnt, wu, ww, U)

    slab = jax.ShapeDtypeStruct((npad, d), f32)
    sf0, sf1, si0, si1, so0, so1 = pl.kernel(
        functools.partial(_msg_body, rt, km_half),
        out_type=(slab,) * 6,
        mesh=_sc_mesh(),
        scratch_types=[
            pltpu.VMEM((_GB, _LANES), jnp.int32),
            pltpu.VMEM((_GB, _LANES), jnp.int32),
            pltpu.VMEM((_SUB, d), f32),
            pltpu.VMEM((_SUB, d), f32),
            pltpu.VMEM((_SUB, d), f32),
            pltpu.VMEM((_SUB, d), f32),
            pltpu.VMEM_SHARED((npad, d), f32),
        ] + [pltpu.SemaphoreType.DMA] * 8,
    )(src2d, dst2d, zf, zi, zo, zzero)

    blk2 = n // 10
    hn, cn = pl.pallas_call(
        _tc3_body,
        grid=(n // blk2,),
        in_specs=[pl.BlockSpec((blk2, d), lambda i: (i, 0))] * 6 + [
            pl.BlockSpec((_NC, blk2, _LANES), lambda i: (0, i, 0)),
            pl.BlockSpec((blk2, d), lambda i: (i, 0)),
            pl.BlockSpec((blk2, d), lambda i: (i, 0)),
            pl.BlockSpec((1, 3 * d), lambda i: (0, 0)),
        ],
        out_specs=[
            pl.BlockSpec((blk2, d), lambda i: (i, 0)),
            pl.BlockSpec((blk2, d), lambda i: (i, 0)),
        ],
        out_shape=[
            jax.ShapeDtypeStruct((n, d), f32),
            jax.ShapeDtypeStruct((n, d), f32),
        ],
    )(sf0, sf1, si0, si1, so0, so1, dcnt, t, c, bcat)

    return (hn, cn)


# speedup vs baseline: 44.5065x; 1.1736x over previous
import functools

import jax
import jax.numpy as jnp
from jax import lax
from jax.experimental import pallas as pl
from jax.experimental.pallas import tpu as pltpu
from jax.experimental.pallas import tpu_sc as plsc

_LANES = 128
_NS = 16
_NC = 2
_GB = 40
_GBD = 16


def _round_up(a, b):
    return (a + b - 1) // b * b


def _sc_mesh():
    return plsc.VectorSubcoreMesh(core_axis_name="c", subcore_axis_name="s")


def _deg_body(npad, gd, dst2d, out, didx, cnt):
    c = lax.axis_index("c")
    s = lax.axis_index("s")
    wid = c * _NS + s
    zero16 = jnp.zeros((16,), jnp.float32)
    one16 = jnp.ones((16,), jnp.float32)

    def zloop(i, carry):
        cnt[i, :] = zero16
        return carry

    lax.fori_loop(0, npad // 16, zloop, 0)

    def batch(bi, carry):
        g0 = wid * gd + bi * _GBD
        pltpu.sync_copy(dst2d.at[pl.ds(g0, _GBD)], didx)
        for j in range(_GBD):
            for k in range(_LANES // 16):
                idx = didx[j, pl.ds(k * 16, 16)]
                row = lax.shift_right_logical(idx, 4)
                col = lax.bitwise_and(idx, 15)
                plsc.addupdate_scatter(cnt, [row, col], one16)
        return carry

    lax.fori_loop(0, gd // _GBD, batch, 0)
    pltpu.sync_copy(cnt, out.at[c, s])


_DEPTH = 4
_SUB = 64


def _msg_body(rt, km_half,
              src2d, dst2d, zf, zi, zo, zzero,
              sf0, sf1, si0, si1, so0, so1,
              sidx, didx, buf0, buf1, buf2, buf3, acc,
              gsem0, gsem1, gsem2, gsem3, ssem0, ssem1, ssem2, ssem3):
    c = lax.axis_index("c")
    s = lax.axis_index("s")
    rb = s * rt
    bufs = (buf0, buf1, buf2, buf3)
    gsems = (gsem0, gsem1, gsem2, gsem3)
    ssems = (ssem0, ssem1, ssem2, ssem3)
    nq = 2 * _GB

    def run(z_h, init_h, out_h, gbase):
        pltpu.sync_copy(init_h.at[pl.ds(rb, rt)], acc.at[pl.ds(rb, rt)])
        plsc.subcore_barrier()
        tg0 = gbase + s * km_half

        def src_sl(q):
            return z_h.at[sidx.at[q // 2, pl.ds((q % 2) * _SUB, _SUB)]]

        def dst_sl(q):
            return acc.at[didx.at[q // 2, pl.ds((q % 2) * _SUB, _SUB)]]

        def batch(bi, carry):
            g0 = tg0 + bi * _GB
            pltpu.sync_copy(src2d.at[pl.ds(g0, _GB)], sidx)
            pltpu.sync_copy(dst2d.at[pl.ds(g0, _GB)], didx)
            for q in range(_DEPTH - 1):
                pltpu.async_copy(src_sl(q), bufs[q], gsems[q])
            for q in range(nq):
                b = q % _DEPTH
                pltpu.make_async_copy(src_sl(q), bufs[b], gsems[b]).wait()
                pltpu.async_copy(bufs[b], dst_sl(q), ssems[b], add=True)
                if q + _DEPTH - 1 < nq:
                    pb = (q + _DEPTH - 1) % _DEPTH
                    if q >= 1:
                        pltpu.make_async_copy(bufs[pb], dst_sl(q - 1),
                                              ssems[pb]).wait()
                    pltpu.async_copy(src_sl(q + _DEPTH - 1), bufs[pb],
                                     gsems[pb])
            for q in range(nq - _DEPTH, nq):
                b = q % _DEPTH
                pltpu.make_async_copy(bufs[b], dst_sl(q), ssems[b]).wait()
            return carry

        lax.fori_loop(0, km_half // _GB, batch, 0)
        plsc.subcore_barrier()
        pltpu.sync_copy(acc.at[pl.ds(rb, rt)], out_h.at[pl.ds(rb, rt)])

    half_g = _NS * km_half

    @pl.when(c == 0)
    def _():
        run(zf, zf, sf0, 0)
        run(zi, zzero, si1, half_g)
        run(zo, zo, so0, 0)

    @pl.when(c == 1)
    def _():
        run(zi, zi, si0, 0)
        run(zf, zzero, sf1, half_g)
        run(zo, zzero, so1, half_g)


def _tc12_body(xref, href, dref, wuref, wwref, uref, zfref, ziref, zoref, tref,
               dinvref):
    d = 1.0 + jnp.sum(dref[...], axis=(0, 1))[:, None]
    dinv = lax.rsqrt(d)
    dinvref[...] = dinv
    z = (jnp.dot(xref[...], wuref[...], preferred_element_type=jnp.float32)
         + jnp.dot(href[...], wwref[...], preferred_element_type=jnp.float32))
    zs = z * dinv
    dd = xref.shape[1]
    zfref[...] = zs[:, :dd]
    ziref[...] = zs[:, dd:2 * dd]
    zoref[...] = zs[:, 2 * dd:3 * dd]
    tref[...] = jnp.tanh(jnp.dot(xref[...], uref[...],
                                 preferred_element_type=jnp.float32))


def _tc3_body(sf0ref, sf1ref, si0ref, si1ref, so0ref, so1ref,
              dinvref, tref, cref, bref, hout, cout):
    dinv = dinvref[...]
    dd = tref.shape[1]
    bf = bref[0:1, :dd]
    bi = bref[0:1, dd:2 * dd]
    bo = bref[0:1, 2 * dd:3 * dd]
    f = jax.nn.sigmoid((sf0ref[...] + sf1ref[...]) * dinv + bf)
    i = jax.nn.sigmoid(((si0ref[...] + si1ref[...]) * dinv + bi) * tref[...])
    o = jax.nn.sigmoid((so0ref[...] + so1ref[...]) * dinv + bo)
    cn = cref[...] * f + i
    cout[...] = cn
    hout[...] = o * jnp.tanh(cn)


def kernel(x, h, c, edge_index, Wfu, bfu, Wfw, bfw, Wiu, biu, Wiw, biw,
           Wou, bou, Wow, bow, U):
    f32 = jnp.float32
    n, d = x.shape
    e = edge_index.shape[1]
    npad = _round_up(n + 1, _NS * 80)
    rt = npad // _NS
    epad = _round_up(e, _LANES * _NS * 2 * _GB)
    g = epad // _LANES
    km_half = g // (2 * _NS)
    gd = g // (_NS * _NC)

    wu = jnp.concatenate([Wfu, Wiu, Wou], axis=1)
    ww = jnp.concatenate([Wfw, Wiw, Wow], axis=1)
    bcat = jnp.concatenate([bfu + bfw, biu + biw, bou + bow])[None, :]
    xp = jnp.zeros((npad, d), f32).at[:n, :].set(x)
    hp = jnp.zeros((npad, d), f32).at[:n, :].set(h)
    pidx = jnp.arange(epad - e, dtype=jnp.int32)
    pad_src = (pidx * 131) % n
    pad_dst = n + (pidx % (npad - n))
    src2d = jnp.concatenate([edge_index[0], pad_src]).reshape(g, _LANES)
    dst2d = jnp.concatenate([edge_index[1], pad_dst]).reshape(g, _LANES)
    zzero = jnp.zeros((npad, d), f32)

    dcnt = pl.kernel(
        functools.partial(_deg_body, npad, gd),
        out_type=jax.ShapeDtypeStruct((_NC, _NS, npad // 16, 16), f32),
        mesh=_sc_mesh(),
        scratch_types=[
            pltpu.VMEM((_GBD, _LANES), jnp.int32),
            pltpu.VMEM((npad // 16, 16), f32),
        ],
        compiler_params=pltpu.CompilerParams(needs_layout_passes=False),
    )(dst2d)
    dcnt = dcnt.reshape(_NC, _NS, npad)

    blk = npad // 10
    zf, zi, zo, t, dinv = pl.pallas_call(
        _tc12_body,
        grid=(npad // blk,),
        in_specs=[
            pl.BlockSpec((blk, d), lambda i: (i, 0)),
            pl.BlockSpec((blk, d), lambda i: (i, 0)),
            pl.BlockSpec((_NC, _NS, blk), lambda i: (0, 0, i)),
            pl.BlockSpec((d, 3 * d), lambda i: (0, 0)),
            pl.BlockSpec((d, 3 * d), lambda i: (0, 0)),
            pl.BlockSpec((d, d), lambda i: (0, 0)),
        ],
        out_specs=[
            pl.BlockSpec((blk, d), lambda i: (i, 0)),
            pl.BlockSpec((blk, d), lambda i: (i, 0)),
            pl.BlockSpec((blk, d), lambda i: (i, 0)),
            pl.BlockSpec((blk, d), lambda i: (i, 0)),
            pl.BlockSpec((blk, 1), lambda i: (i, 0)),
        ],
        out_shape=[
            jax.ShapeDtypeStruct((npad, d), f32),
            jax.ShapeDtypeStruct((npad, d), f32),
            jax.ShapeDtypeStruct((npad, d), f32),
            jax.ShapeDtypeStruct((npad, d), f32),
            jax.ShapeDtypeStruct((npad, 1), f32),
        ],
    )(xp, hp, dcnt, wu, ww, U)

    slab = jax.ShapeDtypeStruct((npad, d), f32)
    sf0, sf1, si0, si1, so0, so1 = pl.kernel(
        functools.partial(_msg_body, rt, km_half),
        out_type=(slab,) * 6,
        mesh=_sc_mesh(),
        scratch_types=[
            pltpu.VMEM((_GB, _LANES), jnp.int32),
            pltpu.VMEM((_GB, _LANES), jnp.int32),
            pltpu.VMEM((_SUB, d), f32),
            pltpu.VMEM((_SUB, d), f32),
            pltpu.VMEM((_SUB, d), f32),
            pltpu.VMEM((_SUB, d), f32),
            pltpu.VMEM_SHARED((npad, d), f32),
        ] + [pltpu.SemaphoreType.DMA] * 8,
    )(src2d, dst2d, zf, zi, zo, zzero)

    blk2 = n // 10
    hn, cn = pl.pallas_call(
        _tc3_body,
        grid=(n // blk2,),
        in_specs=[pl.BlockSpec((blk2, d), lambda i: (i, 0))] * 6 + [
            pl.BlockSpec((blk2, 1), lambda i: (i, 0)),
            pl.BlockSpec((blk2, d), lambda i: (i, 0)),
            pl.BlockSpec((blk2, d), lambda i: (i, 0)),
            pl.BlockSpec((1, 3 * d), lambda i: (0, 0)),
        ],
        out_specs=[
            pl.BlockSpec((blk2, d), lambda i: (i, 0)),
            pl.BlockSpec((blk2, d), lambda i: (i, 0)),
        ],
        out_shape=[
            jax.ShapeDtypeStruct((n, d), f32),
            jax.ShapeDtypeStruct((n, d), f32),
        ],
    )(sf0, sf1, si0, si1, so0, so1, dinv, t, c, bcat)

    return (hn, cn)


# speedup vs baseline: 48.3187x; 1.0857x over previous
import functools

import jax
import jax.numpy as jnp
from jax import lax
from jax.experimental import pallas as pl
from jax.experimental.pallas import tpu as pltpu
from jax.experimental.pallas import tpu_sc as plsc

_LANES = 128
_NS = 16
_NC = 2
_GB = 40
_GBD = 16


def _round_up(a, b):
    return (a + b - 1) // b * b


def _sc_mesh():
    return plsc.VectorSubcoreMesh(core_axis_name="c", subcore_axis_name="s")


def _deg_body(npad, gd, dst2d, out, didx, cnt):
    c = lax.axis_index("c")
    s = lax.axis_index("s")
    wid = c * _NS + s
    zero16 = jnp.zeros((16,), jnp.float32)
    one16 = jnp.ones((16,), jnp.float32)

    def zloop(i, carry):
        cnt[i, :] = zero16
        return carry

    lax.fori_loop(0, npad // 16, zloop, 0)

    def batch(bi, carry):
        g0 = wid * gd + bi * _GBD
        pltpu.sync_copy(dst2d.at[pl.ds(g0, _GBD)], didx)
        for j in range(_GBD):
            for k in range(_LANES // 16):
                idx = didx[j, pl.ds(k * 16, 16)]
                row = lax.shift_right_logical(idx, 4)
                col = lax.bitwise_and(idx, 15)
                plsc.addupdate_scatter(cnt, [row, col], one16)
        return carry

    lax.fori_loop(0, gd // _GBD, batch, 0)
    pltpu.sync_copy(cnt, out.at[c, s])


_DEPTH = 8
_SUB = 32


def _msg_body(rt, km_half,
              src2d, dst2d, zf, zi, zo, zzero,
              sf0, sf1, si0, si1, so0, so1,
              sidx, didx, *rest):
    c = lax.axis_index("c")
    s = lax.axis_index("s")
    rb = s * rt
    bufs = rest[:_DEPTH]
    acc = rest[_DEPTH]
    gsems = rest[_DEPTH + 1:2 * _DEPTH + 1]
    ssems = rest[2 * _DEPTH + 1:]
    spg = _LANES // _SUB
    nq = spg * _GB

    def run(z_h, init_h, out_h, gbase):
        pltpu.sync_copy(init_h.at[pl.ds(rb, rt)], acc.at[pl.ds(rb, rt)])
        plsc.subcore_barrier()
        tg0 = gbase + s * km_half

        def src_sl(q):
            return z_h.at[sidx.at[q // spg, pl.ds((q % spg) * _SUB, _SUB)]]

        def dst_sl(q):
            return acc.at[didx.at[q // spg, pl.ds((q % spg) * _SUB, _SUB)]]

        def batch(bi, carry):
            g0 = tg0 + bi * _GB
            pltpu.sync_copy(src2d.at[pl.ds(g0, _GB)], sidx)
            pltpu.sync_copy(dst2d.at[pl.ds(g0, _GB)], didx)
            for q in range(_DEPTH - 1):
                pltpu.async_copy(src_sl(q), bufs[q], gsems[q])
            for q in range(nq):
                b = q % _DEPTH
                pltpu.make_async_copy(src_sl(q), bufs[b], gsems[b]).wait()
                pltpu.async_copy(bufs[b], dst_sl(q), ssems[b], add=True)
                if q + _DEPTH - 1 < nq:
                    pb = (q + _DEPTH - 1) % _DEPTH
                    if q >= 1:
                        pltpu.make_async_copy(bufs[pb], dst_sl(q - 1),
                                              ssems[pb]).wait()
                    pltpu.async_copy(src_sl(q + _DEPTH - 1), bufs[pb],
                                     gsems[pb])
            for q in range(nq - _DEPTH, nq):
                b = q % _DEPTH
                pltpu.make_async_copy(bufs[b], dst_sl(q), ssems[b]).wait()
            return carry

        lax.fori_loop(0, km_half // _GB, batch, 0)
        plsc.subcore_barrier()
        pltpu.sync_copy(acc.at[pl.ds(rb, rt)], out_h.at[pl.ds(rb, rt)])

    half_g = _NS * km_half

    @pl.when(c == 0)
    def _():
        run(zf, zf, sf0, 0)
        run(zi, zzero, si1, half_g)
        run(zo, zo, so0, 0)

    @pl.when(c == 1)
    def _():
        run(zi, zi, si0, 0)
        run(zf, zzero, sf1, half_g)
        run(zo, zzero, so1, half_g)


def _tc12_body(xref, href, dref, wuref, wwref, uref, zfref, ziref, zoref, tref,
               dinvref):
    d = 1.0 + jnp.sum(dref[...], axis=(0, 1))[:, None]
    dinv = lax.rsqrt(d)
    dinvref[...] = dinv
    z = (jnp.dot(xref[...], wuref[...], preferred_element_type=jnp.float32)
         + jnp.dot(href[...], wwref[...], preferred_element_type=jnp.float32))
    zs = z * dinv
    dd = xref.shape[1]
    zfref[...] = zs[:, :dd]
    ziref[...] = zs[:, dd:2 * dd]
    zoref[...] = zs[:, 2 * dd:3 * dd]
    tref[...] = jnp.tanh(jnp.dot(xref[...], uref[...],
                                 preferred_element_type=jnp.float32))


def _tc3_body(sf0ref, sf1ref, si0ref, si1ref, so0ref, so1ref,
              dinvref, tref, cref, bref, hout, cout):
    dinv = dinvref[...]
    dd = tref.shape[1]
    bf = bref[0:1, :dd]
    bi = bref[0:1, dd:2 * dd]
    bo = bref[0:1, 2 * dd:3 * dd]
    f = jax.nn.sigmoid((sf0ref[...] + sf1ref[...]) * dinv + bf)
    i = jax.nn.sigmoid(((si0ref[...] + si1ref[...]) * dinv + bi) * tref[...])
    o = jax.nn.sigmoid((so0ref[...] + so1ref[...]) * dinv + bo)
    cn = cref[...] * f + i
    cout[...] = cn
    hout[...] = o * jnp.tanh(cn)


def kernel(x, h, c, edge_index, Wfu, bfu, Wfw, bfw, Wiu, biu, Wiw, biw,
           Wou, bou, Wow, bow, U):
    f32 = jnp.float32
    n, d = x.shape
    e = edge_index.shape[1]
    npad = _round_up(n + 1, _NS * 80)
    rt = npad // _NS
    epad = _round_up(e, _LANES * _NS * 2 * _GB)
    g = epad // _LANES
    km_half = g // (2 * _NS)
    gd = g // (_NS * _NC)

    wu = jnp.concatenate([Wfu, Wiu, Wou], axis=1)
    ww = jnp.concatenate([Wfw, Wiw, Wow], axis=1)
    bcat = jnp.concatenate([bfu + bfw, biu + biw, bou + bow])[None, :]
    xp = jnp.zeros((npad, d), f32).at[:n, :].set(x)
    hp = jnp.zeros((npad, d), f32).at[:n, :].set(h)
    pidx = jnp.arange(epad - e, dtype=jnp.int32)
    pad_src = (pidx * 131) % n
    pad_dst = n + (pidx % (npad - n))
    src2d = jnp.concatenate([edge_index[0], pad_src]).reshape(g, _LANES)
    dst2d = jnp.concatenate([edge_index[1], pad_dst]).reshape(g, _LANES)
    zzero = jnp.zeros((npad, d), f32)

    dcnt = pl.kernel(
        functools.partial(_deg_body, npad, gd),
        out_type=jax.ShapeDtypeStruct((_NC, _NS, npad // 16, 16), f32),
        mesh=_sc_mesh(),
        scratch_types=[
            pltpu.VMEM((_GBD, _LANES), jnp.int32),
            pltpu.VMEM((npad // 16, 16), f32),
        ],
        compiler_params=pltpu.CompilerParams(needs_layout_passes=False),
    )(dst2d)
    dcnt = dcnt.reshape(_NC, _NS, npad)

    blk = npad // 10
    zf, zi, zo, t, dinv = pl.pallas_call(
        _tc12_body,
        grid=(npad // blk,),
        in_specs=[
            pl.BlockSpec((blk, d), lambda i: (i, 0)),
            pl.BlockSpec((blk, d), lambda i: (i, 0)),
            pl.BlockSpec((_NC, _NS, blk), lambda i: (0, 0, i)),
            pl.BlockSpec((d, 3 * d), lambda i: (0, 0)),
            pl.BlockSpec((d, 3 * d), lambda i: (0, 0)),
            pl.BlockSpec((d, d), lambda i: (0, 0)),
        ],
        out_specs=[
            pl.BlockSpec((blk, d), lambda i: (i, 0)),
            pl.BlockSpec((blk, d), lambda i: (i, 0)),
            pl.BlockSpec((blk, d), lambda i: (i, 0)),
            pl.BlockSpec((blk, d), lambda i: (i, 0)),
            pl.BlockSpec((blk, 1), lambda i: (i, 0)),
        ],
        out_shape=[
            jax.ShapeDtypeStruct((npad, d), f32),
            jax.ShapeDtypeStruct((npad, d), f32),
            jax.ShapeDtypeStruct((npad, d), f32),
            jax.ShapeDtypeStruct((npad, d), f32),
            jax.ShapeDtypeStruct((npad, 1), f32),
        ],
    )(xp, hp, dcnt, wu, ww, U)

    slab = jax.ShapeDtypeStruct((npad, d), f32)
    sf0, sf1, si0, si1, so0, so1 = pl.kernel(
        functools.partial(_msg_body, rt, km_half),
        out_type=(slab,) * 6,
        mesh=_sc_mesh(),
        scratch_types=[
            pltpu.VMEM((_GB, _LANES), jnp.int32),
            pltpu.VMEM((_GB, _LANES), jnp.int32),
        ] + [pltpu.VMEM((_SUB, d), f32)] * _DEPTH + [
            pltpu.VMEM_SHARED((npad, d), f32),
        ] + [pltpu.SemaphoreType.DMA] * (2 * _DEPTH),
    )(src2d, dst2d, zf, zi, zo, zzero)

    blk2 = n // 10
    hn, cn = pl.pallas_call(
        _tc3_body,
        grid=(n // blk2,),
        in_specs=[pl.BlockSpec((blk2, d), lambda i: (i, 0))] * 6 + [
            pl.BlockSpec((blk2, 1), lambda i: (i, 0)),
            pl.BlockSpec((blk2, d), lambda i: (i, 0)),
            pl.BlockSpec((blk2, d), lambda i: (i, 0)),
            pl.BlockSpec((1, 3 * d), lambda i: (0, 0)),
        ],
        out_specs=[
            pl.BlockSpec((blk2, d), lambda i: (i, 0)),
            pl.BlockSpec((blk2, d), lambda i: (i, 0)),
        ],
        out_shape=[
            jax.ShapeDtypeStruct((n, d), f32),
            jax.ShapeDtypeStruct((n, d), f32),
        ],
    )(sf0, sf1, si0, si1, so0, so1, dinv, t, c, bcat)

    return (hn, cn)


# speedup vs baseline: 48.5016x; 1.0038x over previous
import functools

import jax
import jax.numpy as jnp
from jax import lax
from jax.experimental import pallas as pl
from jax.experimental.pallas import tpu as pltpu
from jax.experimental.pallas import tpu_sc as plsc

_LANES = 128
_NS = 16
_NC = 2
_GB = 40
_GBD = 16


def _round_up(a, b):
    return (a + b - 1) // b * b


def _sc_mesh():
    return plsc.VectorSubcoreMesh(core_axis_name="c", subcore_axis_name="s")


def _deg_body(npad, gd, dst2d, out, didx, cnt):
    c = lax.axis_index("c")
    s = lax.axis_index("s")
    wid = c * _NS + s
    zero16 = jnp.zeros((16,), jnp.float32)
    one16 = jnp.ones((16,), jnp.float32)

    def zloop(i, carry):
        cnt[i, :] = zero16
        return carry

    lax.fori_loop(0, npad // 16, zloop, 0)

    def batch(bi, carry):
        g0 = wid * gd + bi * _GBD
        pltpu.sync_copy(dst2d.at[pl.ds(g0, _GBD)], didx)
        for j in range(_GBD):
            for k in range(_LANES // 16):
                idx = didx[j, pl.ds(k * 16, 16)]
                row = lax.shift_right_logical(idx, 4)
                col = lax.bitwise_and(idx, 15)
                plsc.addupdate_scatter(cnt, [row, col], one16)
        return carry

    lax.fori_loop(0, gd // _GBD, batch, 0)
    pltpu.sync_copy(cnt, out.at[c, s])


_DEPTH = 8
_SUB = 32


def _msg_body(rt, km_half,
              src2d, dst2d, zf, zi, zo, zzero,
              sf0, sf1, si0, si1, so0, so1,
              sidx, didx, *rest):
    c = lax.axis_index("c")
    s = lax.axis_index("s")
    rb = s * rt
    bufs = rest[:_DEPTH]
    acc = rest[_DEPTH]
    gsems = rest[_DEPTH + 1:2 * _DEPTH + 1]
    ssems = rest[2 * _DEPTH + 1:]
    spg = _LANES // _SUB
    nq = spg * _GB

    def run(z_h, init_h, out_h, gbase):
        pltpu.sync_copy(init_h.at[pl.ds(rb, rt)], acc.at[pl.ds(rb, rt)])
        plsc.subcore_barrier()
        tg0 = gbase + s * km_half

        def src_sl(q):
            return z_h.at[sidx.at[q // spg, pl.ds((q % spg) * _SUB, _SUB)]]

        def dst_sl(q):
            return acc.at[didx.at[q // spg, pl.ds((q % spg) * _SUB, _SUB)]]

        def batch(bi, carry):
            g0 = tg0 + bi * _GB
            pltpu.sync_copy(src2d.at[pl.ds(g0, _GB)], sidx)
            pltpu.sync_copy(dst2d.at[pl.ds(g0, _GB)], didx)
            for q in range(_DEPTH - 1):
                pltpu.async_copy(src_sl(q), bufs[q], gsems[q])
            for q in range(nq):
                b = q % _DEPTH
                pltpu.make_async_copy(src_sl(q), bufs[b], gsems[b]).wait()
                pltpu.async_copy(bufs[b], dst_sl(q), ssems[b], add=True)
                if q + _DEPTH - 1 < nq:
                    pb = (q + _DEPTH - 1) % _DEPTH
                    if q >= 1:
                        pltpu.make_async_copy(bufs[pb], dst_sl(q - 1),
                                              ssems[pb]).wait()
                    pltpu.async_copy(src_sl(q + _DEPTH - 1), bufs[pb],
                                     gsems[pb])
            for q in range(nq - _DEPTH, nq):
                b = q % _DEPTH
                pltpu.make_async_copy(bufs[b], dst_sl(q), ssems[b]).wait()
            return carry

        lax.fori_loop(0, km_half // _GB, batch, 0)
        plsc.subcore_barrier()
        pltpu.sync_copy(acc.at[pl.ds(rb, rt)], out_h.at[pl.ds(rb, rt)])

    half_g = _NS * km_half

    @pl.when(c == 0)
    def _():
        run(zf, zf, sf0, 0)
        run(zi, zzero, si1, half_g)
        run(zo, zo, so0, 0)

    @pl.when(c == 1)
    def _():
        run(zi, zi, si0, 0)
        run(zf, zzero, sf1, half_g)
        run(zo, zzero, so1, half_g)


def _tc12_body(xref, href, dref, wfuref, wfwref, wiuref, wiwref, wouref,
               wowref, uref, zfref, ziref, zoref, tref, dinvref):
    f32 = jnp.float32
    d = 1.0 + jnp.sum(dref[...], axis=(0, 1))[:, None]
    dinv = lax.rsqrt(d)
    dinvref[...] = dinv
    xv = xref[...]
    hv = href[...]
    zfref[...] = (jnp.dot(xv, wfuref[...], preferred_element_type=f32)
                  + jnp.dot(hv, wfwref[...], preferred_element_type=f32)) * dinv
    ziref[...] = (jnp.dot(xv, wiuref[...], preferred_element_type=f32)
                  + jnp.dot(hv, wiwref[...], preferred_element_type=f32)) * dinv
    zoref[...] = (jnp.dot(xv, wouref[...], preferred_element_type=f32)
                  + jnp.dot(hv, wowref[...], preferred_element_type=f32)) * dinv
    tref[...] = jnp.tanh(jnp.dot(xv, uref[...], preferred_element_type=f32))


def _tc3_body(sf0ref, sf1ref, si0ref, si1ref, so0ref, so1ref,
              dinvref, tref, cref, bref, hout, cout):
    dinv = dinvref[...]
    dd = tref.shape[1]
    bf = bref[0:1, :dd]
    bi = bref[0:1, dd:2 * dd]
    bo = bref[0:1, 2 * dd:3 * dd]
    f = jax.nn.sigmoid((sf0ref[...] + sf1ref[...]) * dinv + bf)
    i = jax.nn.sigmoid(((si0ref[...] + si1ref[...]) * dinv + bi) * tref[...])
    o = jax.nn.sigmoid((so0ref[...] + so1ref[...]) * dinv + bo)
    cn = cref[...] * f + i
    cout[...] = cn
    hout[...] = o * jnp.tanh(cn)


def kernel(x, h, c, edge_index, Wfu, bfu, Wfw, bfw, Wiu, biu, Wiw, biw,
           Wou, bou, Wow, bow, U):
    f32 = jnp.float32
    n, d = x.shape
    e = edge_index.shape[1]
    npad = _round_up(n + 1, _NS * 80)
    rt = npad // _NS
    epad = _round_up(e, _LANES * _NS * 2 * _GB)
    g = epad // _LANES
    km_half = g // (2 * _NS)
    gd = g // (_NS * _NC)

    bcat = jnp.concatenate([bfu + bfw, biu + biw, bou + bow])[None, :]
    pidx = jnp.arange(epad - e, dtype=jnp.int32)
    pad_src = (pidx * 131) % n
    pad_dst = n + (pidx % (npad - n))
    src2d = jnp.concatenate([edge_index[0], pad_src]).reshape(g, _LANES)
    dst2d = jnp.concatenate([edge_index[1], pad_dst]).reshape(g, _LANES)
    zzero = jnp.zeros((npad, d), f32)

    dcnt = pl.kernel(
        functools.partial(_deg_body, npad, gd),
        out_type=jax.ShapeDtypeStruct((_NC, _NS, npad // 16, 16), f32),
        mesh=_sc_mesh(),
        scratch_types=[
            pltpu.VMEM((_GBD, _LANES), jnp.int32),
            pltpu.VMEM((npad // 16, 16), f32),
        ],
        compiler_params=pltpu.CompilerParams(needs_layout_passes=False),
    )(dst2d)
    dcnt = dcnt.reshape(_NC, _NS, npad)

    blk = npad // 10
    zf, zi, zo, t, dinv = pl.pallas_call(
        _tc12_body,
        grid=(npad // blk,),
        in_specs=[
            pl.BlockSpec((blk, d), lambda i: (i, 0)),
            pl.BlockSpec((blk, d), lambda i: (i, 0)),
            pl.BlockSpec((_NC, _NS, blk), lambda i: (0, 0, i)),
        ] + [pl.BlockSpec((d, d), lambda i: (0, 0))] * 7,
        out_specs=[
            pl.BlockSpec((blk, d), lambda i: (i, 0)),
            pl.BlockSpec((blk, d), lambda i: (i, 0)),
            pl.BlockSpec((blk, d), lambda i: (i, 0)),
            pl.BlockSpec((blk, d), lambda i: (i, 0)),
            pl.BlockSpec((blk, 1), lambda i: (i, 0)),
        ],
        out_shape=[
            jax.ShapeDtypeStruct((npad, d), f32),
            jax.ShapeDtypeStruct((npad, d), f32),
            jax.ShapeDtypeStruct((npad, d), f32),
            jax.ShapeDtypeStruct((npad, d), f32),
            jax.ShapeDtypeStruct((npad, 1), f32),
        ],
    )(x, h, dcnt, Wfu, Wfw, Wiu, Wiw, Wou, Wow, U)

    slab = jax.ShapeDtypeStruct((npad, d), f32)
    sf0, sf1, si0, si1, so0, so1 = pl.kernel(
        functools.partial(_msg_body, rt, km_half),
        out_type=(slab,) * 6,
        mesh=_sc_mesh(),
        scratch_types=[
            pltpu.VMEM((_GB, _LANES), jnp.int32),
            pltpu.VMEM((_GB, _LANES), jnp.int32),
        ] + [pltpu.VMEM((_SUB, d), f32)] * _DEPTH + [
            pltpu.VMEM_SHARED((npad, d), f32),
        ] + [pltpu.SemaphoreType.DMA] * (2 * _DEPTH),
    )(src2d, dst2d, zf, zi, zo, zzero)

    blk2 = n // 10
    hn, cn = pl.pallas_call(
        _tc3_body,
        grid=(n // blk2,),
        in_specs=[pl.BlockSpec((blk2, d), lambda i: (i, 0))] * 6 + [
            pl.BlockSpec((blk2, 1), lambda i: (i, 0)),
            pl.BlockSpec((blk2, d), lambda i: (i, 0)),
            pl.BlockSpec((blk2, d), lambda i: (i, 0)),
            pl.BlockSpec((1, 3 * d), lambda i: (0, 0)),
        ],
        out_specs=[
            pl.BlockSpec((blk2, d), lambda i: (i, 0)),
            pl.BlockSpec((blk2, d), lambda i: (i, 0)),
        ],
        out_shape=[
            jax.ShapeDtypeStruct((n, d), f32),
            jax.ShapeDtypeStruct((n, d), f32),
        ],
    )(sf0, sf1, si0, si1, so0, so1, dinv, t, c, bcat)

    return (hn, cn)


# speedup vs baseline: 50.6732x; 1.0448x over previous
import functools

import jax
import jax.numpy as jnp
from jax import lax
from jax.experimental import pallas as pl
from jax.experimental.pallas import tpu as pltpu
from jax.experimental.pallas import tpu_sc as plsc

_LANES = 128
_NS = 16
_NC = 2
_GB = 40
_GBD = 16


def _round_up(a, b):
    return (a + b - 1) // b * b


def _sc_mesh():
    return plsc.VectorSubcoreMesh(core_axis_name="c", subcore_axis_name="s")


def _deg_body(npad, gd, dst2d, out, didx, cnt):
    c = lax.axis_index("c")
    s = lax.axis_index("s")
    wid = c * _NS + s
    zero16 = jnp.zeros((16,), jnp.float32)
    one16 = jnp.ones((16,), jnp.float32)

    def zloop(i, carry):
        cnt[i, :] = zero16
        return carry

    lax.fori_loop(0, npad // 16, zloop, 0)

    def batch(bi, carry):
        g0 = wid * gd + bi * _GBD
        pltpu.sync_copy(dst2d.at[pl.ds(g0, _GBD)], didx)
        for j in range(_GBD):
            for k in range(_LANES // 16):
                idx = didx[j, pl.ds(k * 16, 16)]
                row = lax.shift_right_logical(idx, 4)
                col = lax.bitwise_and(idx, 15)
                plsc.addupdate_scatter(cnt, [row, col], one16)
        return carry

    lax.fori_loop(0, gd // _GBD, batch, 0)
    pltpu.sync_copy(cnt, out.at[c, s])


_DEPTH = 8
_SUB = 32


def _msg_body(rt, km_half,
              src2d, dst2d, zf, zi, zo, zzero,
              sf, si, so0, so1,
              sidx, didx, *rest):
    c = lax.axis_index("c")
    s = lax.axis_index("s")
    rb = s * rt
    bufs = rest[:_DEPTH]
    acc = rest[_DEPTH]
    gsems = rest[_DEPTH + 1:2 * _DEPTH + 1]
    ssems = rest[2 * _DEPTH + 1:]
    spg = _LANES // _SUB
    nq = spg * _GB

    def run(z_h, init_h, out_h, gbase, km):
        pltpu.sync_copy(init_h.at[pl.ds(rb, rt)], acc.at[pl.ds(rb, rt)])
        plsc.subcore_barrier()
        tg0 = gbase + s * km

        def src_sl(q):
            return z_h.at[sidx.at[q // spg, pl.ds((q % spg) * _SUB, _SUB)]]

        def dst_sl(q):
            return acc.at[didx.at[q // spg, pl.ds((q % spg) * _SUB, _SUB)]]

        def batch(bi, carry):
            g0 = tg0 + bi * _GB
            pltpu.sync_copy(src2d.at[pl.ds(g0, _GB)], sidx)
            pltpu.sync_copy(dst2d.at[pl.ds(g0, _GB)], didx)
            for q in range(_DEPTH - 1):
                pltpu.async_copy(src_sl(q), bufs[q], gsems[q])
            for q in range(nq):
                b = q % _DEPTH
                pltpu.make_async_copy(src_sl(q), bufs[b], gsems[b]).wait()
                pltpu.async_copy(bufs[b], dst_sl(q), ssems[b], add=True)
                if q + _DEPTH - 1 < nq:
                    pb = (q + _DEPTH - 1) % _DEPTH
                    if q >= 1:
                        pltpu.make_async_copy(bufs[pb], dst_sl(q - 1),
                                              ssems[pb]).wait()
                    pltpu.async_copy(src_sl(q + _DEPTH - 1), bufs[pb],
                                     gsems[pb])
            for q in range(nq - _DEPTH, nq):
                b = q % _DEPTH
                pltpu.make_async_copy(bufs[b], dst_sl(q), ssems[b]).wait()
            return carry

        lax.fori_loop(0, km // _GB, batch, 0)
        plsc.subcore_barrier()
        pltpu.sync_copy(acc.at[pl.ds(rb, rt)], out_h.at[pl.ds(rb, rt)])

    half_g = _NS * km_half

    @pl.when(c == 0)
    def _():
        run(zf, zf, sf, 0, 2 * km_half)
        run(zo, zo, so0, 0, km_half)

    @pl.when(c == 1)
    def _():
        run(zi, zi, si, 0, 2 * km_half)
        run(zo, zzero, so1, half_g, km_half)


def _tc12_body(xref, href, dref, wfuref, wfwref, wiuref, wiwref, wouref,
               wowref, zfref, ziref, zoref, dinvref):
    f32 = jnp.float32
    d = 1.0 + jnp.sum(dref[...], axis=(0, 1))[:, None]
    dinv = lax.rsqrt(d)
    dinvref[...] = dinv
    xv = xref[...]
    hv = href[...]
    zfref[...] = (jnp.dot(xv, wfuref[...], preferred_element_type=f32)
                  + jnp.dot(hv, wfwref[...], preferred_element_type=f32)) * dinv
    ziref[...] = (jnp.dot(xv, wiuref[...], preferred_element_type=f32)
                  + jnp.dot(hv, wiwref[...], preferred_element_type=f32)) * dinv
    zoref[...] = (jnp.dot(xv, wouref[...], preferred_element_type=f32)
                  + jnp.dot(hv, wowref[...], preferred_element_type=f32)) * dinv


def _tc3_body(sfref, siref, so0ref, so1ref,
              dinvref, xref, uref, cref, bref, hout, cout):
    f32 = jnp.float32
    dinv = dinvref[...]
    dd = xref.shape[1]
    bf = bref[0:1, :dd]
    bi = bref[0:1, dd:2 * dd]
    bo = bref[0:1, 2 * dd:3 * dd]
    t = jnp.tanh(jnp.dot(xref[...], uref[...], preferred_element_type=f32))
    f = jax.nn.sigmoid(sfref[...] * dinv + bf)
    i = jax.nn.sigmoid((siref[...] * dinv + bi) * t)
    o = jax.nn.sigmoid((so0ref[...] + so1ref[...]) * dinv + bo)
    cn = cref[...] * f + i
    cout[...] = cn
    hout[...] = o * jnp.tanh(cn)


def kernel(x, h, c, edge_index, Wfu, bfu, Wfw, bfw, Wiu, biu, Wiw, biw,
           Wou, bou, Wow, bow, U):
    f32 = jnp.float32
    n, d = x.shape
    e = edge_index.shape[1]
    npad = _round_up(n + 1, _NS * 80)
    rt = npad // _NS
    epad = _round_up(e, _LANES * _NS * 2 * _GB)
    g = epad // _LANES
    km_half = g // (2 * _NS)
    gd = g // (_NS * _NC)

    bcat = jnp.concatenate([bfu + bfw, biu + biw, bou + bow])[None, :]
    pidx = jnp.arange(epad - e, dtype=jnp.int32)
    pad_src = (pidx * 131) % n
    pad_dst = n + (pidx % (npad - n))
    src2d = jnp.concatenate([edge_index[0], pad_src]).reshape(g, _LANES)
    dst2d = jnp.concatenate([edge_index[1], pad_dst]).reshape(g, _LANES)
    zzero = jnp.zeros((npad, d), f32)

    dcnt = pl.kernel(
        functools.partial(_deg_body, npad, gd),
        out_type=jax.ShapeDtypeStruct((_NC, _NS, npad // 16, 16), f32),
        mesh=_sc_mesh(),
        scratch_types=[
            pltpu.VMEM((_GBD, _LANES), jnp.int32),
            pltpu.VMEM((npad // 16, 16), f32),
        ],
        compiler_params=pltpu.CompilerParams(needs_layout_passes=False),
    )(dst2d)
    dcnt = dcnt.reshape(_NC, _NS, npad)

    blk = npad // 10
    zf, zi, zo, dinv = pl.pallas_call(
        _tc12_body,
        grid=(npad // blk,),
        in_specs=[
            pl.BlockSpec((blk, d), lambda i: (i, 0)),
            pl.BlockSpec((blk, d), lambda i: (i, 0)),
            pl.BlockSpec((_NC, _NS, blk), lambda i: (0, 0, i)),
        ] + [pl.BlockSpec((d, d), lambda i: (0, 0))] * 6,
        out_specs=[
            pl.BlockSpec((blk, d), lambda i: (i, 0)),
            pl.BlockSpec((blk, d), lambda i: (i, 0)),
            pl.BlockSpec((blk, d), lambda i: (i, 0)),
            pl.BlockSpec((blk, 1), lambda i: (i, 0)),
        ],
        out_shape=[
            jax.ShapeDtypeStruct((npad, d), f32),
            jax.ShapeDtypeStruct((npad, d), f32),
            jax.ShapeDtypeStruct((npad, d), f32),
            jax.ShapeDtypeStruct((npad, 1), f32),
        ],
    )(x, h, dcnt, Wfu, Wfw, Wiu, Wiw, Wou, Wow)

    slab = jax.ShapeDtypeStruct((npad, d), f32)
    sf, si, so0, so1 = pl.kernel(
        functools.partial(_msg_body, rt, km_half),
        out_type=(slab,) * 4,
        mesh=_sc_mesh(),
        scratch_types=[
            pltpu.VMEM((_GB, _LANES), jnp.int32),
            pltpu.VMEM((_GB, _LANES), jnp.int32),
        ] + [pltpu.VMEM((_SUB, d), f32)] * _DEPTH + [
            pltpu.VMEM_SHARED((npad, d), f32),
        ] + [pltpu.SemaphoreType.DMA] * (2 * _DEPTH),
    )(src2d, dst2d, zf, zi, zo, zzero)

    blk2 = n // 10
    hn, cn = pl.pallas_call(
        _tc3_body,
        grid=(n // blk2,),
        in_specs=[pl.BlockSpec((blk2, d), lambda i: (i, 0))] * 4 + [
            pl.BlockSpec((blk2, 1), lambda i: (i, 0)),
            pl.BlockSpec((blk2, d), lambda i: (i, 0)),
            pl.BlockSpec((d, d), lambda i: (0, 0)),
            pl.BlockSpec((blk2, d), lambda i: (i, 0)),
            pl.BlockSpec((1, 3 * d), lambda i: (0, 0)),
        ],
        out_specs=[
            pl.BlockSpec((blk2, d), lambda i: (i, 0)),
            pl.BlockSpec((blk2, d), lambda i: (i, 0)),
        ],
        out_shape=[
            jax.ShapeDtypeStruct((n, d), f32),
            jax.ShapeDtypeStruct((n, d), f32),
        ],
    )(sf, si, so0, so1, dinv, x, U, c, bcat)

    return (hn, cn)
